# baseline (speedup 1.0000x reference)
import jax
import jax.numpy as jnp
from jax import lax
from jax.experimental import pallas as pl
from jax.experimental.pallas import tpu as pltpu
from jax.experimental.pallas import tpu_sc as plsc

_D = 64
_SEQ = 200
_BATCH = 4096
_SCALE = float(_D) ** 0.5

_NC = 2
_NS = 16
_NW = _NC * _NS
_LANES = 16

_BCHUNK = 256
_UPS = _BATCH // _BCHUNK
_NUNITS = _SEQ * _UPS
_UPW = _NUNITS // _NW
_TPITCH = _BCHUNK + 1

_NR = 3
_NT = 2
_GROUP = 6
_SLOTS = 102

_SH, _BH, _SL, _BL = _SEQ // 8, _BATCH // 128, 8, 128


def _sc_body(x_hbm, table_hbm, pe_hbm, out_hbm, pe_v,
             i0, i1, i2, r0, r1, r2, t0, t1,
             si0, si1, si2, sg0, sg1, sg2, ss0, ss1):
    idx = [i0, i1, i2]
    rows = [r0, r1, r2]
    tbuf = [t0, t1]
    isem = [si0, si1, si2]
    gsem = [sg0, sg1, sg2]
    ssem = [ss0, ss1]
    wid = lax.axis_index("s") * _NC + lax.axis_index("c")
    u0 = wid * _UPW

    pltpu.sync_copy(pe_hbm, pe_v)

    iota = lax.iota(jnp.int32, _LANES)
    idx_d = [iota + g * _LANES for g in range(_D // _LANES)]

    def i_start(k, b):
        u = u0 + k
        s = u // _UPS
        c = u % _UPS
        for h in range(2):
            pltpu.async_copy(
                x_hbm.at[s // _SL, 2 * c + h, s % _SL, :],
                idx[b].at[pl.ds(h * _BL, _BL)], isem[b])

    def i_wait(k, b):
        u = u0 + k
        s = u // _UPS
        c = u % _UPS
        for h in range(2):
            pltpu.make_async_copy(
                x_hbm.at[s // _SL, 2 * c + h, s % _SL, :],
                idx[b].at[pl.ds(h * _BL, _BL)], isem[b]).wait()

    def g_start(k, b):
        pltpu.async_copy(table_hbm.at[idx[b]], rows[b], gsem[b])

    def g_wait(k, b):
        pltpu.make_async_copy(table_hbm.at[idx[b]], rows[b], gsem[b]).wait()

    def out_slice(k):
        u = u0 + k
        s = u // _UPS
        b0 = (u % _UPS) * _BCHUNK
        return out_hbm.at[s, :, pl.ds(b0, _BCHUNK)]

    def s_start(k, b):
        pltpu.async_copy(tbuf[b].at[:, pl.ds(0, _BCHUNK)], out_slice(k), ssem[b])

    def s_wait(k, b):
        pltpu.make_async_copy(tbuf[b].at[:, pl.ds(0, _BCHUNK)], out_slice(k),
                              ssem[b]).wait()

    def compute(k, rb, tb):
        u = u0 + k
        s = u // _UPS
        pe_g = [pe_v[s, pl.ds(g * _LANES, _LANES)] for g in range(_D // _LANES)]

        @plsc.parallel_loop(0, _BCHUNK, unroll=4)
        def t_body(t, rb=rb, tb=tb, pe_g=pe_g):
            tvec = jnp.full((_LANES,), t, jnp.int32)
            for g in range(_D // _LANES):
                vals = rows[rb][t, pl.ds(g * _LANES, _LANES)]
                plsc.store_scatter(tbuf[tb], [idx_d[g], tvec],
                                   vals * _SCALE + pe_g[g])

    for b in range(_NR):
        i_start(b, b)
    for b in range(_NT):
        i_wait(b, b)
        g_start(b, b)

    def group(i, carry):
        for b6 in range(_GROUP):
            k = i * _GROUP + b6
            rb = b6 % _NR
            tb = b6 % _NT

            @pl.when(k < _UPW)
            def _(k=k, rb=rb, tb=tb):
                g_wait(k, rb)

                @pl.when(k + _NR < _UPW)
                def _():
                    i_start(k + _NR, rb)

                @pl.when(k + _NT < _UPW)
                def _(b6=b6):
                    i_wait(k + _NT, (b6 + _NT) % _NR)
                    g_start(k + _NT, (b6 + _NT) % _NR)

                @pl.when(k >= _NT)
                def _():
                    s_wait(k - _NT, tb)
                compute(k, rb, tb)
                s_start(k, tb)
        return carry

    lax.fori_loop(0, _SLOTS // _GROUP, group, 0)

    for b in range(_NT):
        k = _UPW - _NT + b
        s_wait(k, k % _NT)


@jax.jit
def _embed(x_tiles, table, pe_seq):
    mesh = plsc.VectorSubcoreMesh(core_axis_name="c", subcore_axis_name="s")
    launch = pl.kernel(
        _sc_body,
        out_type=jax.ShapeDtypeStruct((_SEQ, _D, _BATCH), jnp.float32),
        mesh=mesh,
        scratch_types=(
            [pltpu.VMEM((_SEQ, _D), jnp.float32)]
            + [pltpu.VMEM((_BCHUNK,), jnp.int32)] * _NR
            + [pltpu.VMEM((_BCHUNK, _D), jnp.float32)] * _NR
            + [pltpu.VMEM((_D, _TPITCH), jnp.float32)] * _NT
            + [pltpu.SemaphoreType.DMA] * (_NR + _NR + _NT)
        ),
        compiler_params=pltpu.CompilerParams(use_tc_tiling_on_sc=False,
                                             needs_layout_passes=False),
    )
    return launch(x_tiles, table, pe_seq)


def kernel(x, table, pe):
    x_tiles = jnp.transpose(
        x.astype(jnp.int32).reshape(_BH, _BL, _SH, _SL), (2, 0, 3, 1))
    pe_seq = pe[:_SEQ].astype(jnp.float32)
    out_sdb = _embed(x_tiles, table, pe_seq)
    return jnp.transpose(out_sdb, (2, 0, 1))


# speedup vs baseline: 1.2741x; 1.2741x over previous
import jax
import jax.numpy as jnp
from jax import lax
from jax.experimental import pallas as pl
from jax.experimental.pallas import tpu as pltpu
from jax.experimental.pallas import tpu_sc as plsc

_D = 64
_SEQ = 200
_BATCH = 4096
_SCALE = float(_D) ** 0.5

_NC = 2
_NS = 16
_NW = _NC * _NS
_LANES = 16

_BCHUNK = 256
_UPS = _BATCH // _BCHUNK
_NUNITS = _SEQ * _UPS
_UPW = _NUNITS // _NW
_TPITCH = _BCHUNK + 1

_NR = 3
_NT = 2
_GROUP = 6
_SLOTS = 102

_SH, _BH, _SL, _BL = _SEQ // 8, _BATCH // 128, 8, 128


def _sc_body(x_hbm, table_hbm, pe_hbm, out_hbm, pe_v,
             i0, i1, i2, r0, r1, r2, tp, t5s,
             si0, si1, si2, sg0, sg1, sg2, sl0, sl1, ss0, ss1):
    idx = [i0, i1, i2]
    rows = [r0, r1, r2]
    sid = lax.axis_index("s")
    isem = [si0, si1, si2]
    gsem = [sg0, sg1, sg2]
    lsem = [sl0, sl1]
    ssem = [ss0, ss1]
    wid = lax.axis_index("s") * _NC + lax.axis_index("c")
    u0 = wid * _UPW

    pltpu.sync_copy(pe_hbm, pe_v)

    iota = lax.iota(jnp.int32, _LANES)
    idx_tr = [(iota + g * _LANES) // 8 for g in range(_D // _LANES)]
    idx_rl = [(iota + g * _LANES) % 8 for g in range(_D // _LANES)]

    def i_start(k, b):
        u = u0 + k
        s = u // _UPS
        c = u % _UPS
        for h in range(2):
            pltpu.async_copy(
                x_hbm.at[s // _SL, 2 * c + h, s % _SL, :],
                idx[b].at[pl.ds(h * _BL, _BL)], isem[b])

    def i_wait(k, b):
        u = u0 + k
        s = u // _UPS
        c = u % _UPS
        for h in range(2):
            pltpu.make_async_copy(
                x_hbm.at[s // _SL, 2 * c + h, s % _SL, :],
                idx[b].at[pl.ds(h * _BL, _BL)], isem[b]).wait()

    def g_start(k, b):
        pltpu.async_copy(table_hbm.at[idx[b]], rows[b], gsem[b])

    def g_wait(k, b):
        pltpu.make_async_copy(table_hbm.at[idx[b]], rows[b], gsem[b]).wait()

    def l_start(k, b):
        for tc in range(2):
            pltpu.async_copy(tp.at[:, :, pl.ds(tc * _BL, _BL)],
                             t5s.at[sid, b, :, tc], lsem[b])

    def l_wait(k, b):
        for tc in range(2):
            pltpu.make_async_copy(tp.at[:, :, pl.ds(tc * _BL, _BL)],
                                  t5s.at[sid, b, :, tc], lsem[b]).wait()

    def out_slice(k):
        u = u0 + k
        s = u // _UPS
        c = u % _UPS
        return out_hbm.at[s, :, pl.ds(2 * c, 2), :, :]

    def s_start(k, b):
        pltpu.async_copy(t5s.at[sid, b], out_slice(k), ssem[b])

    def s_wait(k, b):
        pltpu.make_async_copy(t5s.at[sid, b], out_slice(k), ssem[b]).wait()

    def compute(k, rb):
        u = u0 + k
        s = u // _UPS
        pe_g = [pe_v[s, pl.ds(g * _LANES, _LANES)] for g in range(_D // _LANES)]

        @plsc.parallel_loop(0, _BCHUNK, unroll=4)
        def t_body(t, rb=rb, pe_g=pe_g):
            tvec = jnp.full((_LANES,), t, jnp.int32)
            for g in range(_D // _LANES):
                vals = rows[rb][t, pl.ds(g * _LANES, _LANES)]
                plsc.store_scatter(tp, [idx_tr[g], idx_rl[g], tvec],
                                   vals * _SCALE + pe_g[g])

    for b in range(_NR):
        i_start(b, b)
    for b in range(_NT):
        i_wait(b, b)
        g_start(b, b)

    def group(i, carry):
        for b6 in range(_GROUP):
            k = i * _GROUP + b6
            rb = b6 % _NR
            tb = b6 % _NT

            @pl.when(k < _UPW)
            def _(k=k, b6=b6, rb=rb, tb=tb):
                g_wait(k, rb)

                @pl.when(k + _NR < _UPW)
                def _():
                    i_start(k + _NR, rb)

                @pl.when(k + _NT < _UPW)
                def _(b6=b6):
                    i_wait(k + _NT, (b6 + _NT) % _NR)
                    g_start(k + _NT, (b6 + _NT) % _NR)

                @pl.when(k >= _NT)
                def _(tb=tb):
                    s_wait(k - _NT, tb)

                @pl.when(k >= 1)
                def _(b6=b6):
                    pb = (b6 - 1) % _NT
                    l_wait(k - 1, pb)
                    s_start(k - 1, pb)
                compute(k, rb)
                l_start(k, tb)
        return carry

    lax.fori_loop(0, _SLOTS // _GROUP, group, 0)

    kl = _UPW - 1
    l_wait(kl, kl % _NT)
    s_start(kl, kl % _NT)
    s_wait(_UPW - 2, (_UPW - 2) % _NT)
    s_wait(kl, kl % _NT)


@jax.jit
def _embed(x_tiles, table, pe_seq):
    mesh = plsc.VectorSubcoreMesh(core_axis_name="c", subcore_axis_name="s")
    launch = pl.kernel(
        _sc_body,
        out_type=jax.ShapeDtypeStruct((_SEQ, _D // 8, _BATCH // _BL, 8, _BL),
                                      jnp.float32),
        mesh=mesh,
        scratch_types=(
            [pltpu.VMEM((_SEQ, _D), jnp.float32)]
            + [pltpu.VMEM((_BCHUNK,), jnp.int32)] * _NR
            + [pltpu.VMEM((_BCHUNK, _D), jnp.float32)] * _NR
            + [pltpu.VMEM((8, 8, _TPITCH), jnp.float32)]
            + [pltpu.VMEM_SHARED((_NS, _NT, 8, 2, 8, _BL), jnp.float32)]
            + [pltpu.SemaphoreType.DMA] * (_NR + _NR + _NT + _NT)
        ),
        compiler_params=pltpu.CompilerParams(use_tc_tiling_on_sc=False,
                                             needs_layout_passes=False),
    )
    return launch(x_tiles, table, pe_seq)


def kernel(x, table, pe):
    x_tiles = jnp.transpose(
        x.astype(jnp.int32).reshape(_BH, _BL, _SH, _SL), (2, 0, 3, 1))
    pe_seq = pe[:_SEQ].astype(jnp.float32)
    out5 = _embed(x_tiles, table, pe_seq)
    out = jnp.transpose(out5, (2, 4, 0, 1, 3)).reshape(_BATCH, _SEQ, _D)
    return out
